# baseline (speedup 1.0000x reference)
import functools

import jax
import jax.numpy as jnp
from jax import lax
from jax.experimental import pallas as pl
from jax.experimental.pallas import tpu as pltpu
from jax.experimental.pallas import tpu_sc as plsc

BATCH = 16384
HIST = 200
EMBED = 32
TOKENS = 100000

NC = 2
NS = 16
NW = NC * NS
B_PER_W = BATCH // NW
HALF = HIST // 2
NBUF = 8

_mesh = plsc.VectorSubcoreMesh(core_axis_name="c", subcore_axis_name="s")


@functools.partial(
    pl.kernel,
    mesh=_mesh,
    out_type=jax.ShapeDtypeStruct((BATCH, EMBED), jnp.float32),
    scratch_types=[
        pltpu.VMEM((NBUF, 2, HALF), jnp.int32),
        pltpu.VMEM((NBUF, 2, HALF, EMBED), jnp.float32),
        pltpu.VMEM((B_PER_W, EMBED), jnp.float32),
        pltpu.SemaphoreType.DMA,
        [pltpu.SemaphoreType.DMA] * NBUF,
    ],
    compiler_params=pltpu.CompilerParams(use_tc_tiling_on_sc=False),
)
def _sc_embed_sum(x_hbm, table_hbm, out_hbm, idx_v, rows_v, out_v,
                  isem, gsems):
    cid = lax.axis_index("c")
    sid = lax.axis_index("s")
    wid = sid * NC + cid
    base = wid * B_PER_W

    def fire_idx(r, p):
        pltpu.async_copy(x_hbm.at[base + r], idx_v.at[p], isem)

    def wait_idx(r, p):
        pltpu.make_async_copy(x_hbm.at[base + r], idx_v.at[p], isem).wait()

    def fire_gather(p):
        pltpu.async_copy(table_hbm.at[idx_v.at[p, 0]], rows_v.at[p, 0], gsems[p])
        pltpu.async_copy(table_hbm.at[idx_v.at[p, 1]], rows_v.at[p, 1], gsems[p])

    def wait_gather(p):
        pltpu.make_async_copy(table_hbm.at[idx_v.at[p, 0]], rows_v.at[p, 0],
                              gsems[p]).wait()
        pltpu.make_async_copy(table_hbm.at[idx_v.at[p, 1]], rows_v.at[p, 1],
                              gsems[p]).wait()

    def accumulate(p, r):
        def hbody(h, accs):
            a00, a01, a10, a11 = accs
            a00 = a00 + rows_v[p, 0, h, pl.ds(0, 16)]
            a10 = a10 + rows_v[p, 0, h, pl.ds(16, 16)]
            a01 = a01 + rows_v[p, 1, h, pl.ds(0, 16)]
            a11 = a11 + rows_v[p, 1, h, pl.ds(16, 16)]
            return (a00, a01, a10, a11)

        zero = jnp.zeros((16,), jnp.float32)
        a00, a01, a10, a11 = lax.fori_loop(
            0, HALF, hbody, (zero, zero, zero, zero), unroll=10)
        out_v[r, pl.ds(0, 16)] = a00 + a01
        out_v[r, pl.ds(16, 16)] = a10 + a11

    for j in range(NBUF):
        fire_idx(j, j)
    for j in range(NBUF - 1):
        wait_idx(j, j)
        fire_gather(j)

    @pl.loop(0, B_PER_W, step=NBUF)
    def ring_body(r0):
        for b in range(NBUF):
            r = r0 + b

            @pl.when(r + NBUF - 1 < B_PER_W)
            def _():
                wait_idx(r + NBUF - 1, (b + NBUF - 1) % NBUF)
                fire_gather((b + NBUF - 1) % NBUF)

            wait_gather(b)

            @pl.when(r + NBUF < B_PER_W)
            def _():
                fire_idx(r + NBUF, b)

            accumulate(b, r)

    pltpu.sync_copy(out_v, out_hbm.at[pl.ds(base, B_PER_W)])


def kernel(x, action_emb):
    x3 = x.astype(jnp.int32).reshape(BATCH, 2, HALF)
    return _sc_embed_sum(x3, action_emb)


# speedup vs baseline: 1.0010x; 1.0010x over previous
import functools

import jax
import jax.numpy as jnp
from jax import lax
from jax.experimental import pallas as pl
from jax.experimental.pallas import tpu as pltpu
from jax.experimental.pallas import tpu_sc as plsc

BATCH = 16384
HIST = 200
EMBED = 32
TOKENS = 100000

NC = 2
NS = 16
NW = NC * NS
B_PER_W = BATCH // NW
HALF = HIST // 2
NBUF = 4

_mesh = plsc.VectorSubcoreMesh(core_axis_name="c", subcore_axis_name="s")


@functools.partial(
    pl.kernel,
    mesh=_mesh,
    out_type=jax.ShapeDtypeStruct((BATCH, EMBED), jnp.float32),
    scratch_types=[
        pltpu.VMEM((NBUF, 2, HALF), jnp.int32),
        pltpu.VMEM((NBUF, 2, HALF, EMBED), jnp.float32),
        pltpu.VMEM((B_PER_W, EMBED), jnp.float32),
        pltpu.SemaphoreType.DMA,
        [pltpu.SemaphoreType.DMA] * NBUF,
    ],
    compiler_params=pltpu.CompilerParams(use_tc_tiling_on_sc=False),
)
def _sc_embed_sum(x_hbm, table_hbm, out_hbm, idx_v, rows_v, out_v,
                  isem, gsems):
    cid = lax.axis_index("c")
    sid = lax.axis_index("s")
    wid = sid * NC + cid
    base = wid * B_PER_W

    def fire_idx(r, p):
        pltpu.async_copy(x_hbm.at[base + r], idx_v.at[p], isem)

    def wait_idx(r, p):
        pltpu.make_async_copy(x_hbm.at[base + r], idx_v.at[p], isem).wait()

    def fire_gather(p):
        pltpu.async_copy(table_hbm.at[idx_v.at[p, 0]], rows_v.at[p, 0], gsems[p])
        pltpu.async_copy(table_hbm.at[idx_v.at[p, 1]], rows_v.at[p, 1], gsems[p])

    def wait_gather(p):
        pltpu.make_async_copy(table_hbm.at[idx_v.at[p, 0]], rows_v.at[p, 0],
                              gsems[p]).wait()
        pltpu.make_async_copy(table_hbm.at[idx_v.at[p, 1]], rows_v.at[p, 1],
                              gsems[p]).wait()

    def accumulate(p, r):
        def hbody(h, accs):
            a00, a01, a10, a11 = accs
            a00 = a00 + rows_v[p, 0, h, pl.ds(0, 16)]
            a10 = a10 + rows_v[p, 0, h, pl.ds(16, 16)]
            a01 = a01 + rows_v[p, 1, h, pl.ds(0, 16)]
            a11 = a11 + rows_v[p, 1, h, pl.ds(16, 16)]
            return (a00, a01, a10, a11)

        zero = jnp.zeros((16,), jnp.float32)
        a00, a01, a10, a11 = lax.fori_loop(
            0, HALF, hbody, (zero, zero, zero, zero), unroll=10)
        out_v[r, pl.ds(0, 16)] = a00 + a01
        out_v[r, pl.ds(16, 16)] = a10 + a11

    for j in range(NBUF):
        fire_idx(j, j)
    for j in range(NBUF - 1):
        wait_idx(j, j)
        fire_gather(j)

    @pl.loop(0, B_PER_W, step=NBUF)
    def ring_body(r0):
        for b in range(NBUF):
            r = r0 + b

            @pl.when(r + NBUF - 1 < B_PER_W)
            def _():
                wait_idx(r + NBUF - 1, (b + NBUF - 1) % NBUF)
                fire_gather((b + NBUF - 1) % NBUF)

            wait_gather(b)

            @pl.when(r + NBUF < B_PER_W)
            def _():
                fire_idx(r + NBUF, b)

            accumulate(b, r)

    pltpu.sync_copy(out_v, out_hbm.at[pl.ds(base, B_PER_W)])


def kernel(x, action_emb):
    x3 = x.astype(jnp.int32).reshape(BATCH, 2, HALF)
    return _sc_embed_sum(x3, action_emb)
